# baseline (speedup 1.0000x reference)
import functools

import jax
import jax.numpy as jnp
from jax.experimental import pallas as pl
from jax.experimental.pallas import tpu as pltpu


def _round_up(x: int, m: int) -> int:
    return (x + m - 1) // m * m


def _matmul_tanh(mean_tok, w_ref, b_ref, out_dtype):
    y = jax.lax.dot_general(
        mean_tok.astype(w_ref.dtype), w_ref[...],
        dimension_numbers=(((1,), (1,)), ((), ())),
        preferred_element_type=jnp.float32)
    return jnp.tanh(y + b_ref[...].astype(jnp.float32)).astype(out_dtype)


def _split_epilogue_block(x_ref, w_ref, b_ref, o_ref, acc_ref, y_ref, sem,
                          *, inv_s, bt, tpc):
    c = pl.program_id(0)
    s = pl.program_id(1)
    rows = bt * tpc
    n_head = rows - bt
    mean_tok = jnp.sum(x_ref[...], axis=1, dtype=jnp.float32) * inv_s
    acc_ref[pl.ds(s * bt, bt), :] = mean_tok

    @pl.when(s == tpc - 2)
    def _head_epilogue():
        y_ref[pl.ds(0, n_head), :] = _matmul_tanh(
            acc_ref[pl.ds(0, n_head), :], w_ref, b_ref, y_ref.dtype)
        pltpu.make_async_copy(
            y_ref.at[pl.ds(0, n_head), :],
            o_ref.at[pl.ds(c * rows, n_head), :],
            sem).start()

    @pl.when(s == tpc - 1)
    def _tail_epilogue():
        y_ref[pl.ds(n_head, bt), :] = _matmul_tanh(
            mean_tok, w_ref, b_ref, y_ref.dtype)
        tail_copy = pltpu.make_async_copy(
            y_ref.at[pl.ds(n_head, bt), :],
            o_ref.at[pl.ds(c * rows + n_head, bt), :],
            sem)
        tail_copy.start()
        pltpu.make_async_copy(
            y_ref.at[pl.ds(0, n_head), :],
            o_ref.at[pl.ds(c * rows, n_head), :],
            sem).wait()
        tail_copy.wait()


def _fused_epilogue_block(x_ref, w_ref, b_ref, o_ref, *, inv_s):
    mean_tok = jnp.sum(x_ref[...], axis=1, dtype=jnp.float32) * inv_s
    o_ref[...] = _matmul_tanh(mean_tok, w_ref, b_ref, o_ref.dtype)


def kernel(hidden_states, weight, bias):
    B, S, H = hidden_states.shape
    out_dtype = hidden_states.dtype
    x_isz = hidden_states.dtype.itemsize

    row_bytes = S * H * x_isz
    budget = 7 << 20
    Bt = max(8, min(128, (budget // max(1, row_bytes)) // 8 * 8))
    if B <= 8:
        Bt = B
    else:
        Bt = min(Bt, max(8, _round_up(pl.cdiv(B, 4), 8)))
    nb = pl.cdiv(B, Bt)

    bias2d = bias.reshape(1, H)
    cost = pl.CostEstimate(
        flops=int(B * S * H + 2 * B * H * H + B * H),
        transcendentals=int(B * H),
        bytes_accessed=int(hidden_states.size * x_isz + weight.size * 4
                           + bias.size * 4 + B * H * out_dtype.itemsize))

    if nb % 2 == 0 and nb >= 6 and B == nb * Bt:
        tpc = nb // 2
        body = functools.partial(_split_epilogue_block,
                                 inv_s=1.0 / S, bt=Bt, tpc=tpc)
        return pl.pallas_call(
            body,
            out_shape=jax.ShapeDtypeStruct((B, H), out_dtype),
            grid=(2, tpc),
            in_specs=[
                pl.BlockSpec((Bt, S, H), lambda c, s: (c * tpc + s, 0, 0)),
                pl.BlockSpec((H, H), lambda c, s: (0, 0)),
                pl.BlockSpec((1, H), lambda c, s: (0, 0)),
            ],
            out_specs=pl.BlockSpec(memory_space=pl.ANY),
            scratch_shapes=[pltpu.VMEM((Bt * (nb // 2), H), jnp.float32),
                            pltpu.VMEM((Bt * (nb // 2), H), out_dtype),
                            pltpu.SemaphoreType.DMA],
            compiler_params=pltpu.CompilerParams(
                dimension_semantics=("parallel", "arbitrary")),
            cost_estimate=cost,
        )(hidden_states, weight, bias2d)

    body = functools.partial(_fused_epilogue_block, inv_s=1.0 / S)
    return pl.pallas_call(
        body,
        out_shape=jax.ShapeDtypeStruct((B, H), out_dtype),
        grid=(nb,),
        in_specs=[
            pl.BlockSpec((Bt, S, H), lambda b: (b, 0, 0)),
            pl.BlockSpec((H, H), lambda b: (0, 0)),
            pl.BlockSpec((1, H), lambda b: (0, 0)),
        ],
        out_specs=pl.BlockSpec((Bt, H), lambda b: (b, 0)),
        compiler_params=pltpu.CompilerParams(
            dimension_semantics=("parallel",)),
        cost_estimate=cost,
    )(hidden_states, weight, bias2d)
